# baseline (speedup 1.0000x reference)
import jax
import jax.numpy as jnp
from jax import lax
from jax.experimental import pallas as pl
from jax.experimental.pallas import tpu as pltpu
from jax.experimental.pallas import tpu_sc as plsc

N = 250
IN = 2048
OUT = 256
R = 7
E = 16000

EP = 16384
NW = 32
EPW = EP // NW
CH = 128
NCH = EPW // CH
ROWS = (R + 1) * N
OUTW = OUT // 2

EB = 8
EPB = EP // EB

NB = 25
NPB = N // NB


def _pack_halves(v):
    k = v.shape[1] // 2
    lo = v[:, :k].astype(jnp.bfloat16).astype(jnp.float32)
    hi = v[:, k:].astype(jnp.bfloat16).astype(jnp.float32)
    lo_w = lax.shift_right_logical(lax.bitcast_convert_type(lo, jnp.int32), 16)
    hi_w = lax.bitwise_and(lax.bitcast_convert_type(hi, jnp.int32),
                           jnp.int32(-65536))
    return lax.bitwise_or(lo_w, hi_w)


def _unpack_halves(w):
    lo = lax.bitcast_convert_type(lax.shift_left(w, 16), jnp.float32)
    hi = lax.bitcast_convert_type(
        lax.bitwise_and(w, jnp.int32(-65536)), jnp.float32)
    return lo, hi


KS = 2
KH = IN // KS


def _mm_body(x_ref, w_ref, root_ref, h_ref, acc_ref):
    r = pl.program_id(0)
    k = pl.program_id(1)

    def accum(b_ref):
        part = jnp.dot(x_ref[...].astype(jnp.bfloat16),
                       b_ref.astype(jnp.bfloat16),
                       preferred_element_type=jnp.float32)

        @pl.when(k == 0)
        def _():
            acc_ref[...] = part

        @pl.when(k > 0)
        def _():
            acc_ref[...] += part

    @pl.when(r < R)
    def _():
        accum(w_ref[0])

    @pl.when(r == R)
    def _():
        accum(root_ref[...])

    @pl.when(k == KS - 1)
    def _():
        h_ref[0] = _pack_halves(acc_ref[...])


def _stage1(x, W, root):
    return pl.pallas_call(
        _mm_body,
        grid=(R + 1, KS),
        in_specs=[
            pl.BlockSpec((N, KH), lambda r, k: (0, k)),
            pl.BlockSpec((1, KH, OUT), lambda r, k: (jnp.minimum(r, R - 1), k, 0)),
            pl.BlockSpec((KH, OUT), lambda r, k: (k, 0)),
        ],
        out_specs=pl.BlockSpec((1, N, OUTW), lambda r, k: (r, 0, 0)),
        out_shape=jax.ShapeDtypeStruct((R + 1, N, OUTW), jnp.int32),
        scratch_shapes=[pltpu.VMEM((N, OUT), jnp.float32)],
    )(x, W, root)


def _sc_body(h_hbm, edges_hbm, m_hbm,
             edges_v, mrow_0, mrow_1, mrow_2, mrow_3,
             rows_a, rows_b, rows_c, rows_d,
             gsa, gsb, gsc, gsd, wsa, wsb, wsc, wsd):
    mrows = (mrow_0, mrow_1, mrow_2, mrow_3)
    cid = lax.axis_index("c")
    sid = lax.axis_index("s")
    base = (cid * 16 + sid) * EPW

    pltpu.sync_copy(edges_hbm.at[:, pl.ds(base, EPW)], edges_v)

    for i in range(EPW // 16):
        s = edges_v[0, pl.ds(i * 16, 16)]
        r = edges_v[1, pl.ds(i * 16, 16)]
        j, k = divmod(i, CH // 16)
        mrows[j][pl.ds(k * 16, 16)] = r * N + s

    def dst(j):
        return m_hbm.at[pl.ds(base + j * CH, CH)]

    bufs = (rows_a, rows_b, rows_c, rows_d)
    gsems = (gsa, gsb, gsc, gsd)
    wsems = (wsa, wsb, wsc, wsd)
    gs = [pltpu.async_copy(h_hbm.at[mrows[j]], bufs[j], gsems[j])
          for j in range(NCH)]
    ws = []
    for j in range(NCH):
        gs[j].wait()
        ws.append(pltpu.async_copy(bufs[j], dst(j), wsems[j]))
    for w in ws:
        w.wait()


def _sc_gather(h_tab, edges):
    mesh = plsc.VectorSubcoreMesh(core_axis_name="c", subcore_axis_name="s")
    f = pl.kernel(
        _sc_body,
        out_type=jax.ShapeDtypeStruct((EP, OUTW), jnp.int32),
        mesh=mesh,
        scratch_types=[
            pltpu.VMEM((2, EPW), jnp.int32),
            pltpu.VMEM((CH,), jnp.int32),
            pltpu.VMEM((CH,), jnp.int32),
            pltpu.VMEM((CH,), jnp.int32),
            pltpu.VMEM((CH,), jnp.int32),
            pltpu.VMEM((CH, OUTW), jnp.int32),
            pltpu.VMEM((CH, OUTW), jnp.int32),
            pltpu.VMEM((CH, OUTW), jnp.int32),
            pltpu.VMEM((CH, OUTW), jnp.int32),
            pltpu.SemaphoreType.DMA,
            pltpu.SemaphoreType.DMA,
            pltpu.SemaphoreType.DMA,
            pltpu.SemaphoreType.DMA,
            pltpu.SemaphoreType.DMA,
            pltpu.SemaphoreType.DMA,
            pltpu.SemaphoreType.DMA,
            pltpu.SemaphoreType.DMA,
        ],
    )
    return f(h_tab, edges)


def _onehots_t(dst_ref, rel_ref):
    dstoht = (lax.broadcasted_iota(jnp.int32, (N, EPB), 0).astype(jnp.float32)
              == dst_ref[...]).astype(jnp.float32)
    reloht = (lax.broadcasted_iota(jnp.int32, (R + 1, EPB), 0).astype(jnp.float32)
              == rel_ref[...]).astype(jnp.float32)
    return dstoht, reloht


def _cnt_body(dst_ref, rel_ref, cnt_ref):
    b = pl.program_id(0)
    dstoht, reloht = _onehots_t(dst_ref, rel_ref)

    @pl.when(b == 0)
    def _():
        cnt_ref[...] = jnp.zeros_like(cnt_ref)

    cnt_ref[...] += lax.dot_general(
        dstoht, reloht, (((1,), (1,)), ((), ())),
        preferred_element_type=jnp.float32)


def _stage3a(dst_f, rel_f):
    return pl.pallas_call(
        _cnt_body,
        grid=(EB,),
        in_specs=[
            pl.BlockSpec((1, EPB), lambda b: (0, b)),
            pl.BlockSpec((1, EPB), lambda b: (0, b)),
        ],
        out_specs=pl.BlockSpec((N, R + 1), lambda b: (0, 0)),
        out_shape=jax.ShapeDtypeStruct((N, R + 1), jnp.float32),
    )(dst_f, rel_f)


def _agg_body(m_ref, dst_ref, rel_ref, cnt_ref, o_ref):
    b = pl.program_id(0)
    dstoht, reloht = _onehots_t(dst_ref, rel_ref)
    rmask = (lax.broadcasted_iota(jnp.int32, (N, R + 1), 1) < R)
    invt = jnp.where(rmask, 1.0 / jnp.maximum(cnt_ref[...], 1.0), 0.0)
    invselt = jnp.dot(invt, reloht, preferred_element_type=jnp.float32)
    sprimet = (dstoht * invselt).astype(jnp.bfloat16)

    m_lo, m_hi = _unpack_halves(m_ref[...])

    @pl.when(b == 0)
    def _():
        o_ref[...] = jnp.zeros_like(o_ref)

    o_ref[:, :OUTW] += jnp.dot(sprimet, m_lo.astype(jnp.bfloat16),
                               preferred_element_type=jnp.float32)
    o_ref[:, OUTW:] += jnp.dot(sprimet, m_hi.astype(jnp.bfloat16),
                               preferred_element_type=jnp.float32)


def _stage3b(m, dst_f, rel_f, cnt):
    return pl.pallas_call(
        _agg_body,
        grid=(EB,),
        in_specs=[
            pl.BlockSpec((EPB, OUTW), lambda b: (b, 0)),
            pl.BlockSpec((1, EPB), lambda b: (0, b)),
            pl.BlockSpec((1, EPB), lambda b: (0, b)),
            pl.BlockSpec((N, R + 1), lambda b: (0, 0)),
        ],
        out_specs=pl.BlockSpec((N, OUT), lambda b: (0, 0)),
        out_shape=jax.ShapeDtypeStruct((N, OUT), jnp.float32),
    )(m, dst_f, rel_f, cnt)


def _fin_body(osum_ref, hroot_ref, bias_ref, fcw_ref, fcb_ref, y_ref):
    nb = pl.program_id(0)

    hr_lo, hr_hi = _unpack_halves(hroot_ref[0])
    hroot = jnp.concatenate([hr_lo, hr_hi], axis=1)
    o = hroot + bias_ref[...] + osum_ref[0]
    o = jnp.maximum(o, 0.0)

    part = jnp.zeros((1, OUT), jnp.float32)
    for n in range(NPB):
        w_n = fcw_ref[:, n * OUT:(n + 1) * OUT]
        part = part + lax.dot_general(
            o[n:n + 1, :], w_n, (((1,), (1,)), ((), ())),
            preferred_element_type=jnp.float32)

    @pl.when(nb == 0)
    def _():
        y_ref[...] = fcb_ref[...]

    y_ref[...] += part


def _stage3c(osum, hroot, bias2, fc_w, fcb2):
    return pl.pallas_call(
        _fin_body,
        grid=(NB,),
        in_specs=[
            pl.BlockSpec((1, NPB, OUT), lambda nb: (nb, 0, 0)),
            pl.BlockSpec((1, NPB, OUTW), lambda nb: (nb, 0, 0)),
            pl.BlockSpec((1, OUT), lambda nb: (0, 0)),
            pl.BlockSpec((OUT, NPB * OUT), lambda nb: (0, nb)),
            pl.BlockSpec((1, OUT), lambda nb: (0, 0)),
        ],
        out_specs=pl.BlockSpec((1, OUT), lambda nb: (0, 0)),
        out_shape=jax.ShapeDtypeStruct((1, OUT), jnp.float32),
    )(osum, hroot, bias2, fc_w, fcb2)


def kernel(node_x, edge_idx, edge_attr, W, root, bias, fc_w, fc_b):
    x = node_x[0].astype(jnp.float32)
    src = edge_idx[0, 0]
    dst = edge_idx[0, 1]
    rel = edge_attr[0]
    pad = EP - E
    src_p = jnp.concatenate([src, jnp.zeros((pad,), jnp.int32)])
    dst_p = jnp.concatenate([dst, jnp.zeros((pad,), jnp.int32)])
    rel_p = jnp.concatenate([rel, jnp.full((pad,), R, jnp.int32)])
    edges = jnp.stack([src_p, rel_p])
    dst_f = dst_p.astype(jnp.float32).reshape(1, EP)
    rel_f = rel_p.astype(jnp.float32).reshape(1, EP)

    H = _stage1(x, W, root)
    h_tab = H.reshape(ROWS, OUTW)

    m = _sc_gather(h_tab, edges)
    cnt = _stage3a(dst_f, rel_f)
    osum = _stage3b(m, dst_f, rel_f, cnt)

    y = _stage3c(osum.reshape(NB, NPB, OUT), H[R].reshape(NB, NPB, OUTW),
                 bias.reshape(1, OUT), fc_w, fc_b.reshape(1, OUT))
    return y


# speedup vs baseline: 1.1882x; 1.1882x over previous
import jax
import jax.numpy as jnp
from jax import lax
from jax.experimental import pallas as pl
from jax.experimental.pallas import tpu as pltpu
from jax.experimental.pallas import tpu_sc as plsc

N = 250
IN = 2048
OUT = 256
R = 7
E = 16000

EP = 16384
NW = 32
EPW = EP // NW
CH = 128
NCH = EPW // CH
ROWS = (R + 1) * N
OUTW = OUT // 2

EB = 8
EPB = EP // EB

NB = 10
NPB = N // NB


def _pack_halves(v):
    k = v.shape[1] // 2
    lo = v[:, :k].astype(jnp.bfloat16).astype(jnp.float32)
    hi = v[:, k:].astype(jnp.bfloat16).astype(jnp.float32)
    lo_w = lax.shift_right_logical(lax.bitcast_convert_type(lo, jnp.int32), 16)
    hi_w = lax.bitwise_and(lax.bitcast_convert_type(hi, jnp.int32),
                           jnp.int32(-65536))
    return lax.bitwise_or(lo_w, hi_w)


def _unpack_halves(w):
    lo = lax.bitcast_convert_type(lax.shift_left(w, 16), jnp.float32)
    hi = lax.bitcast_convert_type(
        lax.bitwise_and(w, jnp.int32(-65536)), jnp.float32)
    return lo, hi


def _mm_body(x_ref, w_ref, root_ref, h_ref):
    r = pl.program_id(0)

    def mm(b_ref):
        acc = jnp.dot(x_ref[...], b_ref.astype(jnp.bfloat16),
                      preferred_element_type=jnp.float32)
        h_ref[0] = _pack_halves(acc)

    @pl.when(r < R)
    def _():
        mm(w_ref[0])

    @pl.when(r == R)
    def _():
        mm(root_ref[...])


def _stage1(xb, W, root):
    return pl.pallas_call(
        _mm_body,
        grid=(R + 1,),
        in_specs=[
            pl.BlockSpec((N, IN), lambda r: (0, 0)),
            pl.BlockSpec((1, IN, OUT), lambda r: (jnp.minimum(r, R - 1), 0, 0)),
            pl.BlockSpec((IN, OUT), lambda r: (0, 0)),
        ],
        out_specs=pl.BlockSpec((1, N, OUTW), lambda r: (r, 0, 0)),
        out_shape=jax.ShapeDtypeStruct((R + 1, N, OUTW), jnp.int32),
    )(xb, W, root)


def _sc_body(h_hbm, edges_hbm, m_hbm,
             edges_v, mrow_0, mrow_1, mrow_2, mrow_3,
             rows_a, rows_b, rows_c, rows_d,
             gsa, gsb, gsc, gsd, wsa, wsb, wsc, wsd):
    mrows = (mrow_0, mrow_1, mrow_2, mrow_3)
    cid = lax.axis_index("c")
    sid = lax.axis_index("s")
    base = (cid * 16 + sid) * EPW

    pltpu.sync_copy(edges_hbm.at[:, pl.ds(base, EPW)], edges_v)

    for i in range(EPW // 16):
        s = edges_v[0, pl.ds(i * 16, 16)]
        r = edges_v[1, pl.ds(i * 16, 16)]
        j, k = divmod(i, CH // 16)
        mrows[j][pl.ds(k * 16, 16)] = r * N + s

    def dst(j):
        return m_hbm.at[pl.ds(base + j * CH, CH)]

    bufs = (rows_a, rows_b, rows_c, rows_d)
    gsems = (gsa, gsb, gsc, gsd)
    wsems = (wsa, wsb, wsc, wsd)
    gs = [pltpu.async_copy(h_hbm.at[mrows[j]], bufs[j], gsems[j])
          for j in range(NCH)]
    ws = []
    for j in range(NCH):
        gs[j].wait()
        ws.append(pltpu.async_copy(bufs[j], dst(j), wsems[j]))
    for w in ws:
        w.wait()


def _sc_gather(h_tab, edges):
    mesh = plsc.VectorSubcoreMesh(core_axis_name="c", subcore_axis_name="s")
    f = pl.kernel(
        _sc_body,
        out_type=jax.ShapeDtypeStruct((EP, OUTW), jnp.int32),
        mesh=mesh,
        scratch_types=[
            pltpu.VMEM((2, EPW), jnp.int32),
            pltpu.VMEM((CH,), jnp.int32),
            pltpu.VMEM((CH,), jnp.int32),
            pltpu.VMEM((CH,), jnp.int32),
            pltpu.VMEM((CH,), jnp.int32),
            pltpu.VMEM((CH, OUTW), jnp.int32),
            pltpu.VMEM((CH, OUTW), jnp.int32),
            pltpu.VMEM((CH, OUTW), jnp.int32),
            pltpu.VMEM((CH, OUTW), jnp.int32),
            pltpu.SemaphoreType.DMA,
            pltpu.SemaphoreType.DMA,
            pltpu.SemaphoreType.DMA,
            pltpu.SemaphoreType.DMA,
            pltpu.SemaphoreType.DMA,
            pltpu.SemaphoreType.DMA,
            pltpu.SemaphoreType.DMA,
            pltpu.SemaphoreType.DMA,
        ],
    )
    return f(h_tab, edges)


def _onehots_t(dst_ref, rel_ref):
    dstoht = (lax.broadcasted_iota(jnp.int32, (N, EPB), 0).astype(jnp.float32)
              == dst_ref[...]).astype(jnp.float32)
    reloht = (lax.broadcasted_iota(jnp.int32, (R + 1, EPB), 0).astype(jnp.float32)
              == rel_ref[...]).astype(jnp.float32)
    return dstoht, reloht


def _cnt_body(dst_ref, rel_ref, cnt_ref):
    b = pl.program_id(0)
    dstoht, reloht = _onehots_t(dst_ref, rel_ref)

    @pl.when(b == 0)
    def _():
        cnt_ref[...] = jnp.zeros_like(cnt_ref)

    cnt_ref[...] += lax.dot_general(
        dstoht, reloht, (((1,), (1,)), ((), ())),
        preferred_element_type=jnp.float32)


def _stage3a(dst_f, rel_f):
    return pl.pallas_call(
        _cnt_body,
        grid=(EB,),
        in_specs=[
            pl.BlockSpec((1, EPB), lambda b: (0, b)),
            pl.BlockSpec((1, EPB), lambda b: (0, b)),
        ],
        out_specs=pl.BlockSpec((N, R + 1), lambda b: (0, 0)),
        out_shape=jax.ShapeDtypeStruct((N, R + 1), jnp.float32),
    )(dst_f, rel_f)


def _agg_body(m_ref, dst_ref, rel_ref, cnt_ref, o_ref):
    b = pl.program_id(0)
    dstoht, reloht = _onehots_t(dst_ref, rel_ref)
    rmask = (lax.broadcasted_iota(jnp.int32, (N, R + 1), 1) < R)
    invt = jnp.where(rmask, 1.0 / jnp.maximum(cnt_ref[...], 1.0), 0.0)
    invselt = jnp.dot(invt, reloht, preferred_element_type=jnp.float32)
    sprimet = (dstoht * invselt).astype(jnp.bfloat16)

    m_lo, m_hi = _unpack_halves(m_ref[...])

    @pl.when(b == 0)
    def _():
        o_ref[...] = jnp.zeros_like(o_ref)

    o_ref[:, :OUTW] += jnp.dot(sprimet, m_lo.astype(jnp.bfloat16),
                               preferred_element_type=jnp.float32)
    o_ref[:, OUTW:] += jnp.dot(sprimet, m_hi.astype(jnp.bfloat16),
                               preferred_element_type=jnp.float32)


def _stage3b(m, dst_f, rel_f, cnt):
    return pl.pallas_call(
        _agg_body,
        grid=(EB,),
        in_specs=[
            pl.BlockSpec((EPB, OUTW), lambda b: (b, 0)),
            pl.BlockSpec((1, EPB), lambda b: (0, b)),
            pl.BlockSpec((1, EPB), lambda b: (0, b)),
            pl.BlockSpec((N, R + 1), lambda b: (0, 0)),
        ],
        out_specs=pl.BlockSpec((N, OUT), lambda b: (0, 0)),
        out_shape=jax.ShapeDtypeStruct((N, OUT), jnp.float32),
    )(m, dst_f, rel_f, cnt)


def _fin_body(osum_ref, hroot_ref, bias_ref, fcw_ref, fcb_ref, y_ref):
    nb = pl.program_id(0)

    hr_lo, hr_hi = _unpack_halves(hroot_ref[0])
    hroot = jnp.concatenate([hr_lo, hr_hi], axis=1)
    o = hroot + bias_ref[...] + osum_ref[0]
    o = jnp.maximum(o, 0.0)

    part = jnp.zeros((1, OUT), jnp.float32)
    for n in range(NPB):
        w_n = fcw_ref[:, n * OUT:(n + 1) * OUT]
        part = part + lax.dot_general(
            o[n:n + 1, :], w_n, (((1,), (1,)), ((), ())),
            preferred_element_type=jnp.float32)

    @pl.when(nb == 0)
    def _():
        y_ref[...] = fcb_ref[...]

    y_ref[...] += part


def _stage3c(osum, hroot, bias2, fc_w, fcb2):
    return pl.pallas_call(
        _fin_body,
        grid=(NB,),
        in_specs=[
            pl.BlockSpec((1, NPB, OUT), lambda nb: (nb, 0, 0)),
            pl.BlockSpec((1, NPB, OUTW), lambda nb: (nb, 0, 0)),
            pl.BlockSpec((1, OUT), lambda nb: (0, 0)),
            pl.BlockSpec((OUT, NPB * OUT), lambda nb: (0, nb)),
            pl.BlockSpec((1, OUT), lambda nb: (0, 0)),
        ],
        out_specs=pl.BlockSpec((1, OUT), lambda nb: (0, 0)),
        out_shape=jax.ShapeDtypeStruct((1, OUT), jnp.float32),
    )(osum, hroot, bias2, fc_w, fcb2)


def kernel(node_x, edge_idx, edge_attr, W, root, bias, fc_w, fc_b):
    x = node_x[0].astype(jnp.float32)
    src = edge_idx[0, 0]
    dst = edge_idx[0, 1]
    rel = edge_attr[0]
    pad = EP - E
    src_p = jnp.concatenate([src, jnp.zeros((pad,), jnp.int32)])
    dst_p = jnp.concatenate([dst, jnp.zeros((pad,), jnp.int32)])
    rel_p = jnp.concatenate([rel, jnp.full((pad,), R, jnp.int32)])
    edges = jnp.stack([src_p, rel_p])
    dst_f = dst_p.astype(jnp.float32).reshape(1, EP)
    rel_f = rel_p.astype(jnp.float32).reshape(1, EP)

    H = _stage1(x.astype(jnp.bfloat16), W, root)
    h_tab = H.reshape(ROWS, OUTW)

    m = _sc_gather(h_tab, edges)
    cnt = _stage3a(dst_f, rel_f)
    osum = _stage3b(m, dst_f, rel_f, cnt)

    y = _stage3c(osum.reshape(NB, NPB, OUT), H[R].reshape(NB, NPB, OUTW),
                 bias.reshape(1, OUT), fc_w, fc_b.reshape(1, OUT))
    return y


# speedup vs baseline: 1.1917x; 1.0029x over previous
import jax
import jax.numpy as jnp
from jax import lax
from jax.experimental import pallas as pl
from jax.experimental.pallas import tpu as pltpu
from jax.experimental.pallas import tpu_sc as plsc

N = 250
IN = 2048
OUT = 256
R = 7
E = 16000

EP = 16384
NW = 32
EPW = EP // NW
CH = 128
NCH = EPW // CH
ROWS = (R + 1) * N
OUTW = OUT // 2

EB = 8
EPB = EP // EB

NB = 10
NPB = N // NB


def _pack_halves(v):
    k = v.shape[1] // 2
    lo = v[:, :k].astype(jnp.bfloat16).astype(jnp.float32)
    hi = v[:, k:].astype(jnp.bfloat16).astype(jnp.float32)
    lo_w = lax.shift_right_logical(lax.bitcast_convert_type(lo, jnp.int32), 16)
    hi_w = lax.bitwise_and(lax.bitcast_convert_type(hi, jnp.int32),
                           jnp.int32(-65536))
    return lax.bitwise_or(lo_w, hi_w)


def _unpack_halves(w):
    lo = lax.bitcast_convert_type(lax.shift_left(w, 16), jnp.float32)
    hi = lax.bitcast_convert_type(
        lax.bitwise_and(w, jnp.int32(-65536)), jnp.float32)
    return lo, hi


def _mm_body(x_ref, w_ref, root_ref, h_ref):
    r = pl.program_id(0)

    def mm(b_ref):
        acc = jnp.dot(x_ref[...], b_ref.astype(jnp.bfloat16),
                      preferred_element_type=jnp.float32)
        h_ref[0] = _pack_halves(acc)

    @pl.when(r < R)
    def _():
        mm(w_ref[0])

    @pl.when(r == R)
    def _():
        mm(root_ref[...])


def _stage1(xb, W, root):
    return pl.pallas_call(
        _mm_body,
        grid=(R + 1,),
        in_specs=[
            pl.BlockSpec((N, IN), lambda r: (0, 0)),
            pl.BlockSpec((1, IN, OUT), lambda r: (jnp.minimum(r, R - 1), 0, 0)),
            pl.BlockSpec((IN, OUT), lambda r: (0, 0)),
        ],
        out_specs=pl.BlockSpec((1, N, OUTW), lambda r: (r, 0, 0)),
        out_shape=jax.ShapeDtypeStruct((R + 1, N, OUTW), jnp.int32),
    )(xb, W, root)


def _sc_body(h_hbm, edges_hbm, m_hbm,
             edges_v, mrow_0, mrow_1, mrow_2, mrow_3,
             rows_a, rows_b, rows_c, rows_d,
             gsa, gsb, gsc, gsd, wsa, wsb, wsc, wsd):
    mrows = (mrow_0, mrow_1, mrow_2, mrow_3)
    cid = lax.axis_index("c")
    sid = lax.axis_index("s")
    base = (cid * 16 + sid) * EPW

    pltpu.sync_copy(edges_hbm.at[:, pl.ds(base, EPW)], edges_v)

    def dst(j):
        return m_hbm.at[pl.ds(base + j * CH, CH)]

    bufs = (rows_a, rows_b, rows_c, rows_d)
    gsems = (gsa, gsb, gsc, gsd)
    wsems = (wsa, wsb, wsc, wsd)

    gs = []
    for j in range(NCH):
        for k in range(CH // 16):
            i = j * (CH // 16) + k
            s = edges_v[0, pl.ds(i * 16, 16)]
            r = edges_v[1, pl.ds(i * 16, 16)]
            mrows[j][pl.ds(k * 16, 16)] = r * N + s
        gs.append(pltpu.async_copy(h_hbm.at[mrows[j]], bufs[j], gsems[j]))
    ws = []
    for j in range(NCH):
        gs[j].wait()
        ws.append(pltpu.async_copy(bufs[j], dst(j), wsems[j]))
    for w in ws:
        w.wait()


def _sc_gather(h_tab, edges):
    mesh = plsc.VectorSubcoreMesh(core_axis_name="c", subcore_axis_name="s")
    f = pl.kernel(
        _sc_body,
        out_type=jax.ShapeDtypeStruct((EP, OUTW), jnp.int32),
        mesh=mesh,
        scratch_types=[
            pltpu.VMEM((2, EPW), jnp.int32),
            pltpu.VMEM((CH,), jnp.int32),
            pltpu.VMEM((CH,), jnp.int32),
            pltpu.VMEM((CH,), jnp.int32),
            pltpu.VMEM((CH,), jnp.int32),
            pltpu.VMEM((CH, OUTW), jnp.int32),
            pltpu.VMEM((CH, OUTW), jnp.int32),
            pltpu.VMEM((CH, OUTW), jnp.int32),
            pltpu.VMEM((CH, OUTW), jnp.int32),
            pltpu.SemaphoreType.DMA,
            pltpu.SemaphoreType.DMA,
            pltpu.SemaphoreType.DMA,
            pltpu.SemaphoreType.DMA,
            pltpu.SemaphoreType.DMA,
            pltpu.SemaphoreType.DMA,
            pltpu.SemaphoreType.DMA,
            pltpu.SemaphoreType.DMA,
        ],
    )
    return f(h_tab, edges)


def _onehots_t(dst_ref, rel_ref):
    dstoht = (lax.broadcasted_iota(jnp.int32, (N, EPB), 0).astype(jnp.float32)
              == dst_ref[...]).astype(jnp.float32)
    reloht = (lax.broadcasted_iota(jnp.int32, (R + 1, EPB), 0).astype(jnp.float32)
              == rel_ref[...]).astype(jnp.float32)
    return dstoht, reloht


def _cnt_body(dst_ref, rel_ref, cnt_ref):
    b = pl.program_id(0)
    dstoht, reloht = _onehots_t(dst_ref, rel_ref)

    @pl.when(b == 0)
    def _():
        cnt_ref[...] = jnp.zeros_like(cnt_ref)

    cnt_ref[...] += lax.dot_general(
        dstoht, reloht, (((1,), (1,)), ((), ())),
        preferred_element_type=jnp.float32)


def _stage3a(dst_f, rel_f):
    return pl.pallas_call(
        _cnt_body,
        grid=(EB,),
        in_specs=[
            pl.BlockSpec((1, EPB), lambda b: (0, b)),
            pl.BlockSpec((1, EPB), lambda b: (0, b)),
        ],
        out_specs=pl.BlockSpec((N, R + 1), lambda b: (0, 0)),
        out_shape=jax.ShapeDtypeStruct((N, R + 1), jnp.float32),
    )(dst_f, rel_f)


def _agg_body(m_ref, dst_ref, rel_ref, cnt_ref, o_ref):
    b = pl.program_id(0)
    dstoht, reloht = _onehots_t(dst_ref, rel_ref)
    rmask = (lax.broadcasted_iota(jnp.int32, (N, R + 1), 1) < R)
    invt = jnp.where(rmask, 1.0 / jnp.maximum(cnt_ref[...], 1.0), 0.0)
    invselt = jnp.dot(invt, reloht, preferred_element_type=jnp.float32)
    sprimet = (dstoht * invselt).astype(jnp.bfloat16)

    m_lo, m_hi = _unpack_halves(m_ref[...])

    @pl.when(b == 0)
    def _():
        o_ref[...] = jnp.zeros_like(o_ref)

    o_ref[:, :OUTW] += jnp.dot(sprimet, m_lo.astype(jnp.bfloat16),
                               preferred_element_type=jnp.float32)
    o_ref[:, OUTW:] += jnp.dot(sprimet, m_hi.astype(jnp.bfloat16),
                               preferred_element_type=jnp.float32)


def _stage3b(m, dst_f, rel_f, cnt):
    return pl.pallas_call(
        _agg_body,
        grid=(EB,),
        in_specs=[
            pl.BlockSpec((EPB, OUTW), lambda b: (b, 0)),
            pl.BlockSpec((1, EPB), lambda b: (0, b)),
            pl.BlockSpec((1, EPB), lambda b: (0, b)),
            pl.BlockSpec((N, R + 1), lambda b: (0, 0)),
        ],
        out_specs=pl.BlockSpec((N, OUT), lambda b: (0, 0)),
        out_shape=jax.ShapeDtypeStruct((N, OUT), jnp.float32),
    )(m, dst_f, rel_f, cnt)


def _fin_body(osum_ref, hroot_ref, bias_ref, fcw_ref, fcb_ref, y_ref):
    nb = pl.program_id(0)

    hr_lo, hr_hi = _unpack_halves(hroot_ref[0])
    hroot = jnp.concatenate([hr_lo, hr_hi], axis=1)
    o = hroot + bias_ref[...] + osum_ref[0]
    o = jnp.maximum(o, 0.0)

    part = jnp.zeros((1, OUT), jnp.float32)
    for n in range(NPB):
        w_n = fcw_ref[:, n * OUT:(n + 1) * OUT]
        part = part + lax.dot_general(
            o[n:n + 1, :], w_n, (((1,), (1,)), ((), ())),
            preferred_element_type=jnp.float32)

    @pl.when(nb == 0)
    def _():
        y_ref[...] = fcb_ref[...]

    y_ref[...] += part


def _stage3c(osum, hroot, bias2, fc_w, fcb2):
    return pl.pallas_call(
        _fin_body,
        grid=(NB,),
        in_specs=[
            pl.BlockSpec((1, NPB, OUT), lambda nb: (nb, 0, 0)),
            pl.BlockSpec((1, NPB, OUTW), lambda nb: (nb, 0, 0)),
            pl.BlockSpec((1, OUT), lambda nb: (0, 0)),
            pl.BlockSpec((OUT, NPB * OUT), lambda nb: (0, nb)),
            pl.BlockSpec((1, OUT), lambda nb: (0, 0)),
        ],
        out_specs=pl.BlockSpec((1, OUT), lambda nb: (0, 0)),
        out_shape=jax.ShapeDtypeStruct((1, OUT), jnp.float32),
    )(osum, hroot, bias2, fc_w, fcb2)


def kernel(node_x, edge_idx, edge_attr, W, root, bias, fc_w, fc_b):
    x = node_x[0].astype(jnp.float32)
    src = edge_idx[0, 0]
    dst = edge_idx[0, 1]
    rel = edge_attr[0]
    pad = EP - E
    src_p = jnp.concatenate([src, jnp.zeros((pad,), jnp.int32)])
    dst_p = jnp.concatenate([dst, jnp.zeros((pad,), jnp.int32)])
    rel_p = jnp.concatenate([rel, jnp.full((pad,), R, jnp.int32)])
    edges = jnp.stack([src_p, rel_p])
    dst_f = dst_p.astype(jnp.float32).reshape(1, EP)
    rel_f = rel_p.astype(jnp.float32).reshape(1, EP)

    H = _stage1(x.astype(jnp.bfloat16), W, root)
    h_tab = H.reshape(ROWS, OUTW)

    m = _sc_gather(h_tab, edges)
    cnt = _stage3a(dst_f, rel_f)
    osum = _stage3b(m, dst_f, rel_f, cnt)

    y = _stage3c(osum.reshape(NB, NPB, OUT), H[R].reshape(NB, NPB, OUTW),
                 bias.reshape(1, OUT), fc_w, fc_b.reshape(1, OUT))
    return y


# speedup vs baseline: 1.1934x; 1.0014x over previous
import jax
import jax.numpy as jnp
from jax import lax
from jax.experimental import pallas as pl
from jax.experimental.pallas import tpu as pltpu
from jax.experimental.pallas import tpu_sc as plsc

N = 250
IN = 2048
OUT = 256
R = 7
E = 16000

EP = 16384
NW = 32
EPW = EP // NW
CH = 64
NCH = EPW // CH
ROWS = (R + 1) * N
OUTW = OUT // 2

EB = 8
EPB = EP // EB

NB = 10
NPB = N // NB


def _pack_halves(v):
    k = v.shape[1] // 2
    lo = v[:, :k].astype(jnp.bfloat16).astype(jnp.float32)
    hi = v[:, k:].astype(jnp.bfloat16).astype(jnp.float32)
    lo_w = lax.shift_right_logical(lax.bitcast_convert_type(lo, jnp.int32), 16)
    hi_w = lax.bitwise_and(lax.bitcast_convert_type(hi, jnp.int32),
                           jnp.int32(-65536))
    return lax.bitwise_or(lo_w, hi_w)


def _unpack_halves(w):
    lo = lax.bitcast_convert_type(lax.shift_left(w, 16), jnp.float32)
    hi = lax.bitcast_convert_type(
        lax.bitwise_and(w, jnp.int32(-65536)), jnp.float32)
    return lo, hi


def _mm_body(x_ref, w_ref, root_ref, h_ref):
    r = pl.program_id(0)

    def mm(b_ref):
        acc = jnp.dot(x_ref[...], b_ref.astype(jnp.bfloat16),
                      preferred_element_type=jnp.float32)
        h_ref[0] = _pack_halves(acc)

    @pl.when(r < R)
    def _():
        mm(w_ref[0])

    @pl.when(r == R)
    def _():
        mm(root_ref[...])


def _stage1(xb, W, root):
    return pl.pallas_call(
        _mm_body,
        grid=(R + 1,),
        in_specs=[
            pl.BlockSpec((N, IN), lambda r: (0, 0)),
            pl.BlockSpec((1, IN, OUT), lambda r: (jnp.minimum(r, R - 1), 0, 0)),
            pl.BlockSpec((IN, OUT), lambda r: (0, 0)),
        ],
        out_specs=pl.BlockSpec((1, N, OUTW), lambda r: (r, 0, 0)),
        out_shape=jax.ShapeDtypeStruct((R + 1, N, OUTW), jnp.int32),
    )(xb, W, root)


def _sc_body(h_hbm, edges_hbm, m_hbm, edges_v, *rest):
    mrows = rest[:NCH]
    bufs = rest[NCH:2 * NCH]
    gsems = rest[2 * NCH:3 * NCH]
    wsems = rest[3 * NCH:4 * NCH]
    cid = lax.axis_index("c")
    sid = lax.axis_index("s")
    base = (cid * 16 + sid) * EPW

    pltpu.sync_copy(edges_hbm.at[:, pl.ds(base, EPW)], edges_v)

    def dst(j):
        return m_hbm.at[pl.ds(base + j * CH, CH)]

    gs = []
    for j in range(NCH):
        for k in range(CH // 16):
            i = j * (CH // 16) + k
            s = edges_v[0, pl.ds(i * 16, 16)]
            r = edges_v[1, pl.ds(i * 16, 16)]
            mrows[j][pl.ds(k * 16, 16)] = r * N + s
        gs.append(pltpu.async_copy(h_hbm.at[mrows[j]], bufs[j], gsems[j]))
    ws = []
    for j in range(NCH):
        gs[j].wait()
        ws.append(pltpu.async_copy(bufs[j], dst(j), wsems[j]))
    for w in ws:
        w.wait()


def _sc_gather(h_tab, edges):
    mesh = plsc.VectorSubcoreMesh(core_axis_name="c", subcore_axis_name="s")
    f = pl.kernel(
        _sc_body,
        out_type=jax.ShapeDtypeStruct((EP, OUTW), jnp.int32),
        mesh=mesh,
        scratch_types=(
            [pltpu.VMEM((2, EPW), jnp.int32)]
            + [pltpu.VMEM((CH,), jnp.int32)] * NCH
            + [pltpu.VMEM((CH, OUTW), jnp.int32)] * NCH
            + [pltpu.SemaphoreType.DMA] * (2 * NCH)
        ),
    )
    return f(h_tab, edges)


def _onehots_t(dst_ref, rel_ref):
    dstoht = (lax.broadcasted_iota(jnp.int32, (N, EPB), 0).astype(jnp.float32)
              == dst_ref[...]).astype(jnp.float32)
    reloht = (lax.broadcasted_iota(jnp.int32, (R + 1, EPB), 0).astype(jnp.float32)
              == rel_ref[...]).astype(jnp.float32)
    return dstoht, reloht


def _cnt_body(dst_ref, rel_ref, cnt_ref):
    b = pl.program_id(0)
    dstoht, reloht = _onehots_t(dst_ref, rel_ref)

    @pl.when(b == 0)
    def _():
        cnt_ref[...] = jnp.zeros_like(cnt_ref)

    cnt_ref[...] += lax.dot_general(
        dstoht, reloht, (((1,), (1,)), ((), ())),
        preferred_element_type=jnp.float32)


def _stage3a(dst_f, rel_f):
    return pl.pallas_call(
        _cnt_body,
        grid=(EB,),
        in_specs=[
            pl.BlockSpec((1, EPB), lambda b: (0, b)),
            pl.BlockSpec((1, EPB), lambda b: (0, b)),
        ],
        out_specs=pl.BlockSpec((N, R + 1), lambda b: (0, 0)),
        out_shape=jax.ShapeDtypeStruct((N, R + 1), jnp.float32),
    )(dst_f, rel_f)


def _agg_body(m_ref, dst_ref, rel_ref, cnt_ref, o_ref):
    b = pl.program_id(0)
    dstoht, reloht = _onehots_t(dst_ref, rel_ref)
    rmask = (lax.broadcasted_iota(jnp.int32, (N, R + 1), 1) < R)
    invt = jnp.where(rmask, 1.0 / jnp.maximum(cnt_ref[...], 1.0), 0.0)
    invselt = jnp.dot(invt, reloht, preferred_element_type=jnp.float32)
    sprimet = (dstoht * invselt).astype(jnp.bfloat16)

    m_lo, m_hi = _unpack_halves(m_ref[...])

    @pl.when(b == 0)
    def _():
        o_ref[...] = jnp.zeros_like(o_ref)

    o_ref[:, :OUTW] += jnp.dot(sprimet, m_lo.astype(jnp.bfloat16),
                               preferred_element_type=jnp.float32)
    o_ref[:, OUTW:] += jnp.dot(sprimet, m_hi.astype(jnp.bfloat16),
                               preferred_element_type=jnp.float32)


def _stage3b(m, dst_f, rel_f, cnt):
    return pl.pallas_call(
        _agg_body,
        grid=(EB,),
        in_specs=[
            pl.BlockSpec((EPB, OUTW), lambda b: (b, 0)),
            pl.BlockSpec((1, EPB), lambda b: (0, b)),
            pl.BlockSpec((1, EPB), lambda b: (0, b)),
            pl.BlockSpec((N, R + 1), lambda b: (0, 0)),
        ],
        out_specs=pl.BlockSpec((N, OUT), lambda b: (0, 0)),
        out_shape=jax.ShapeDtypeStruct((N, OUT), jnp.float32),
    )(m, dst_f, rel_f, cnt)


def _fin_body(osum_ref, hroot_ref, bias_ref, fcw_ref, fcb_ref, y_ref):
    nb = pl.program_id(0)

    hr_lo, hr_hi = _unpack_halves(hroot_ref[0])
    hroot = jnp.concatenate([hr_lo, hr_hi], axis=1)
    o = hroot + bias_ref[...] + osum_ref[0]
    o = jnp.maximum(o, 0.0)

    part = jnp.zeros((1, OUT), jnp.float32)
    for n in range(NPB):
        w_n = fcw_ref[:, n * OUT:(n + 1) * OUT]
        part = part + lax.dot_general(
            o[n:n + 1, :], w_n, (((1,), (1,)), ((), ())),
            preferred_element_type=jnp.float32)

    @pl.when(nb == 0)
    def _():
        y_ref[...] = fcb_ref[...]

    y_ref[...] += part


def _stage3c(osum, hroot, bias2, fc_w, fcb2):
    return pl.pallas_call(
        _fin_body,
        grid=(NB,),
        in_specs=[
            pl.BlockSpec((1, NPB, OUT), lambda nb: (nb, 0, 0)),
            pl.BlockSpec((1, NPB, OUTW), lambda nb: (nb, 0, 0)),
            pl.BlockSpec((1, OUT), lambda nb: (0, 0)),
            pl.BlockSpec((OUT, NPB * OUT), lambda nb: (0, nb)),
            pl.BlockSpec((1, OUT), lambda nb: (0, 0)),
        ],
        out_specs=pl.BlockSpec((1, OUT), lambda nb: (0, 0)),
        out_shape=jax.ShapeDtypeStruct((1, OUT), jnp.float32),
    )(osum, hroot, bias2, fc_w, fcb2)


def kernel(node_x, edge_idx, edge_attr, W, root, bias, fc_w, fc_b):
    x = node_x[0].astype(jnp.float32)
    src = edge_idx[0, 0]
    dst = edge_idx[0, 1]
    rel = edge_attr[0]
    pad = EP - E
    src_p = jnp.concatenate([src, jnp.zeros((pad,), jnp.int32)])
    dst_p = jnp.concatenate([dst, jnp.zeros((pad,), jnp.int32)])
    rel_p = jnp.concatenate([rel, jnp.full((pad,), R, jnp.int32)])
    edges = jnp.stack([src_p, rel_p])
    dst_f = dst_p.astype(jnp.float32).reshape(1, EP)
    rel_f = rel_p.astype(jnp.float32).reshape(1, EP)

    H = _stage1(x.astype(jnp.bfloat16), W, root)
    h_tab = H.reshape(ROWS, OUTW)

    m = _sc_gather(h_tab, edges)
    cnt = _stage3a(dst_f, rel_f)
    osum = _stage3b(m, dst_f, rel_f, cnt)

    y = _stage3c(osum.reshape(NB, NPB, OUT), H[R].reshape(NB, NPB, OUTW),
                 bias.reshape(1, OUT), fc_w, fc_b.reshape(1, OUT))
    return y
